# Initial kernel scaffold; baseline (speedup 1.0000x reference)
import functools
import jax
import jax.numpy as jnp
from jax import lax
from jax.experimental import pallas as pl
from jax.experimental.pallas import tpu as pltpu
from jax.experimental.pallas import tpu_sc as plsc

N_USER = 5000
N_NODES = 10000
D = 128
E = 320000
L = 16
NC = 2
NS = 16
NW = NC * NS
E_PER_TILE = E // NW
B = 80
NBLK = E_PER_TILE // B
DJ = D // L
ZROWS = 125
ROWS_PER_TILE = N_NODES // NS
ZCHUNKS = ROWS_PER_TILE // ZROWS

_mesh = plsc.VectorSubcoreMesh(
    core_axis_name="c", subcore_axis_name="s", num_cores=NC, num_subcores=NS)


@functools.partial(
    pl.kernel,
    out_type=jax.ShapeDtypeStruct((NC, N_NODES, D), jnp.float32),
    mesh=_mesh,
    scratch_types=[
        pltpu.VMEM((NBLK, B), jnp.int32),
        pltpu.VMEM((NBLK, B), jnp.int32),
        pltpu.VMEM((NBLK, B), jnp.float32),
        pltpu.VMEM((B, D), jnp.float32),
        pltpu.VMEM((ZROWS, D), jnp.float32),
        pltpu.VMEM_SHARED((N_NODES, D), jnp.float32),
        pltpu.SemaphoreType.DMA,
    ],
)
def _accumulate(x_hbm, dst_hbm, src_hbm, w_hbm, out_hbm,
                didx, sidx, wbuf, rows, zbuf, acc, sem):
    cid = lax.axis_index("c")
    sid = lax.axis_index("s")
    wid = sid * NC + cid

    pltpu.sync_copy(dst_hbm.at[wid], didx)
    pltpu.sync_copy(src_hbm.at[wid], sidx)
    pltpu.sync_copy(w_hbm.at[wid], wbuf)

    zeros = jnp.zeros((L,), jnp.float32)

    @pl.loop(0, ZROWS)
    def _zero(i):
        for j in range(DJ):
            zbuf[i, pl.ds(j * L, L)] = zeros

    for k in range(ZCHUNKS):
        r0 = sid * ROWS_PER_TILE + k * ZROWS
        pltpu.sync_copy(zbuf, acc.at[pl.ds(r0, ZROWS)])
    plsc.subcore_barrier()

    @pl.loop(0, NBLK)
    def _block(b):
        pltpu.async_copy(x_hbm.at[didx.at[b]], rows, sem).wait()

        @pl.loop(0, B)
        def _edge(e):
            wsp = plsc.load_gather(wbuf.at[b], [jnp.full((L,), e, jnp.int32)])
            for j in range(DJ):
                rows[e, pl.ds(j * L, L)] = rows[e, pl.ds(j * L, L)] * wsp

        pltpu.sync_copy(rows, acc.at[sidx.at[b]], add=True)

    plsc.subcore_barrier()

    for k in range(ZCHUNKS):
        r0 = sid * ROWS_PER_TILE + k * ZROWS
        pltpu.sync_copy(acc.at[pl.ds(r0, ZROWS)], out_hbm.at[cid, pl.ds(r0, ZROWS)])


CROWS = 312
CREM = N_NODES - NW * CROWS


@functools.partial(
    pl.kernel,
    out_type=jax.ShapeDtypeStruct((N_NODES, D), jnp.float32),
    mesh=_mesh,
    scratch_types=[
        pltpu.VMEM((CROWS, D), jnp.float32),
        pltpu.VMEM((CROWS, D), jnp.float32),
        pltpu.VMEM((CREM, D), jnp.float32),
        pltpu.VMEM((CREM, D), jnp.float32),
    ],
)
def _combine(p_hbm, y_hbm, a, b, ra, rb):
    cid = lax.axis_index("c")
    sid = lax.axis_index("s")
    wid = sid * NC + cid
    r0 = wid * CROWS

    pltpu.sync_copy(p_hbm.at[0, pl.ds(r0, CROWS)], a)
    pltpu.sync_copy(p_hbm.at[1, pl.ds(r0, CROWS)], b)

    @pl.loop(0, CROWS)
    def _row(i):
        for j in range(DJ):
            sl = pl.ds(j * L, L)
            a[i, sl] = a[i, sl] + b[i, sl]

    pltpu.sync_copy(a, y_hbm.at[pl.ds(r0, CROWS)])

    @pl.when(wid == 0)
    def _rem():
        r1 = NW * CROWS
        pltpu.sync_copy(p_hbm.at[0, pl.ds(r1, CREM)], ra)
        pltpu.sync_copy(p_hbm.at[1, pl.ds(r1, CREM)], rb)

        @pl.loop(0, CREM)
        def _rrow(i):
            for j in range(DJ):
                sl = pl.ds(j * L, L)
                ra[i, sl] = ra[i, sl] + rb[i, sl]

        pltpu.sync_copy(ra, y_hbm.at[pl.ds(r1, CREM)])


def _layer(x, dst3, src3, w3):
    p = _accumulate(x, dst3, src3, w3)
    return _combine(p)


@jax.jit
def kernel(ini_embeds, edge_index, adj_values):
    src = edge_index[0].astype(jnp.int32).reshape(NW, NBLK, B)
    dst = edge_index[1].astype(jnp.int32).reshape(NW, NBLK, B)
    w = adj_values.reshape(NW, NBLK, B)

    h1 = _layer(ini_embeds, dst, src, w)
    h2 = _layer(h1, dst, src, w)

    tem = jnp.concatenate([h1, h2], axis=-1)
    return tem[:N_USER], tem[N_USER:]


# speedup vs baseline: 3.3015x; 3.3015x over previous
import functools
import jax
import jax.numpy as jnp
from jax import lax
from jax.experimental import pallas as pl
from jax.experimental.pallas import tpu as pltpu
from jax.experimental.pallas import tpu_sc as plsc

N_USER = 5000
N_NODES = 10000
NP = 10240
D = 128
E = 320000
L = 16
NC = 2
NS = 16
NW = NC * NS
E_PER_TILE = E // NW
B = 80
NBLK = E_PER_TILE // B
DJ = D // L
ROWS_PER_TILE = NP // NS
ZCHUNKS = ROWS_PER_TILE // B

_mesh = plsc.VectorSubcoreMesh(
    core_axis_name="c", subcore_axis_name="s", num_cores=NC, num_subcores=NS)


@functools.partial(
    pl.kernel,
    out_type=jax.ShapeDtypeStruct((NC, NP, D), jnp.float32),
    mesh=_mesh,
    scratch_types=[
        pltpu.VMEM((B,), jnp.int32),
        pltpu.VMEM((B,), jnp.int32),
        pltpu.VMEM((B,), jnp.float32),
        pltpu.VMEM((B, D), jnp.float32),
        pltpu.VMEM_SHARED((NP, D), jnp.float32),
        pltpu.SemaphoreType.DMA,
    ],
    compiler_params=pltpu.CompilerParams(needs_layout_passes=False),
)
def _accumulate(x_hbm, dst_hbm, src_hbm, w_hbm, out_hbm,
                didx, sidx, wbuf, rows, acc, sem):
    cid = lax.axis_index("c")
    sid = lax.axis_index("s")
    wid = sid * NC + cid

    zeros = jnp.zeros((L,), jnp.float32)

    @pl.loop(0, B)
    def _zero(i):
        for j in range(DJ):
            rows[i, pl.ds(j * L, L)] = zeros

    for k in range(ZCHUNKS):
        r0 = sid * ROWS_PER_TILE + k * B
        pltpu.sync_copy(rows, acc.at[pl.ds(r0, B)])
    plsc.subcore_barrier()

    @pl.loop(0, NBLK)
    def _block(b):
        pltpu.sync_copy(dst_hbm.at[wid, b], didx)
        pltpu.sync_copy(src_hbm.at[wid, b], sidx)
        pltpu.sync_copy(w_hbm.at[wid, b], wbuf)
        pltpu.async_copy(x_hbm.at[didx], rows, sem).wait()

        @pl.loop(0, B)
        def _edge(e):
            wsp = plsc.load_gather(wbuf, [jnp.full((L,), e, jnp.int32)])
            for j in range(DJ):
                rows[e, pl.ds(j * L, L)] = rows[e, pl.ds(j * L, L)] * wsp

        pltpu.sync_copy(rows, acc.at[sidx], add=True)

    plsc.subcore_barrier()

    for k in range(ZCHUNKS):
        r0 = sid * ROWS_PER_TILE + k * B
        pltpu.sync_copy(acc.at[pl.ds(r0, B)], out_hbm.at[cid, pl.ds(r0, B)])


CROWS = NP // NW


@functools.partial(
    pl.kernel,
    out_type=jax.ShapeDtypeStruct((NP, D), jnp.float32),
    mesh=_mesh,
    scratch_types=[
        pltpu.VMEM((CROWS, D), jnp.float32),
        pltpu.VMEM((CROWS, D), jnp.float32),
    ],
    compiler_params=pltpu.CompilerParams(needs_layout_passes=False),
)
def _combine(p_hbm, y_hbm, a, b):
    cid = lax.axis_index("c")
    sid = lax.axis_index("s")
    wid = sid * NC + cid
    r0 = wid * CROWS

    pltpu.sync_copy(p_hbm.at[0, pl.ds(r0, CROWS)], a)
    pltpu.sync_copy(p_hbm.at[1, pl.ds(r0, CROWS)], b)

    @pl.loop(0, CROWS)
    def _row(i):
        for j in range(DJ):
            sl = pl.ds(j * L, L)
            a[i, sl] = a[i, sl] + b[i, sl]

    pltpu.sync_copy(a, y_hbm.at[pl.ds(r0, CROWS)])


def _layer(x, dst3, src3, w3):
    p = _accumulate(x, dst3, src3, w3)
    return _combine(p)


@jax.jit
def kernel(ini_embeds, edge_index, adj_values):
    src = edge_index[0].astype(jnp.int32).reshape(NW, NBLK, B)
    dst = edge_index[1].astype(jnp.int32).reshape(NW, NBLK, B)
    w = adj_values.reshape(NW, NBLK, B)

    x0 = jnp.zeros((NP, D), jnp.float32).at[:N_NODES].set(ini_embeds)
    h1 = _layer(x0, dst, src, w)
    h2 = _layer(h1, dst, src, w)

    tem = jnp.concatenate([h1[:N_NODES], h2[:N_NODES]], axis=-1)
    return tem[:N_USER], tem[N_USER:]


# speedup vs baseline: 8.3009x; 2.5143x over previous
import functools
import jax
import jax.numpy as jnp
from jax import lax
from jax.experimental import pallas as pl
from jax.experimental.pallas import tpu as pltpu
from jax.experimental.pallas import tpu_sc as plsc

N_USER = 5000
N_NODES = 10000
NP = 10240
D = 128
E = 320000
L = 16
NC = 2
NS = 16
NW = NC * NS
E_PER_TILE = E // NW
B = 80
NBLK = E_PER_TILE // B
CHUNKI = 25
NCHUNK = NBLK // CHUNKI
NPAIR = (CHUNKI - 1) // 2
DJ = D // L
ROWS_PER_TILE = NP // NS
ZCHUNKS = ROWS_PER_TILE // B

_mesh = plsc.VectorSubcoreMesh(
    core_axis_name="c", subcore_axis_name="s", num_cores=NC, num_subcores=NS)


@functools.partial(
    pl.kernel,
    out_type=jax.ShapeDtypeStruct((NC, NP, D), jnp.float32),
    mesh=_mesh,
    scratch_types=[
        pltpu.VMEM((CHUNKI, B), jnp.int32),
        pltpu.VMEM((CHUNKI, B), jnp.int32),
        pltpu.VMEM((CHUNKI, B), jnp.float32),
        pltpu.VMEM((B, D), jnp.float32),
        pltpu.VMEM((B, D), jnp.float32),
        pltpu.VMEM_SHARED((NP, D), jnp.float32),
        pltpu.SemaphoreType.DMA,
        pltpu.SemaphoreType.DMA,
    ],
    compiler_params=pltpu.CompilerParams(needs_layout_passes=False),
)
def _accumulate(x_hbm, dst_hbm, src_hbm, w_hbm, out_hbm,
                didx2, sidx2, wbuf2, rows0, rows1, acc, gsem0, gsem1):
    cid = lax.axis_index("c")
    sid = lax.axis_index("s")
    wid = sid * NC + cid

    rowbufs = (rows0, rows1)
    gsems = (gsem0, gsem1)

    zeros = jnp.zeros((L,), jnp.float32)

    @pl.loop(0, B)
    def _zero(i):
        for j in range(DJ):
            rows0[i, pl.ds(j * L, L)] = zeros

    for k in range(ZCHUNKS):
        r0 = sid * ROWS_PER_TILE + k * B
        pltpu.sync_copy(rows0, acc.at[pl.ds(r0, B)])
    plsc.subcore_barrier()

    def issue_gather(j, s):
        pltpu.async_copy(x_hbm.at[didx2.at[j]], rowbufs[s], gsems[s])

    def wait_gather(s):
        pltpu.make_async_copy(x_hbm.at[pl.ds(0, B)], rowbufs[s], gsems[s]).wait()

    def scale_scatter(j, s):
        rows = rowbufs[s]

        @pl.loop(0, B // L)
        def _grp(g):
            wvec = wbuf2[j, pl.ds(g * L, L)]
            for e in range(L):
                wsp = lax.gather(
                    wvec, jnp.full((L, 1), e, jnp.int32),
                    lax.GatherDimensionNumbers(
                        offset_dims=(), collapsed_slice_dims=(0,),
                        start_index_map=(0,)),
                    (1,), mode=lax.GatherScatterMode.PROMISE_IN_BOUNDS)
                r = g * L + e
                for k in range(DJ):
                    rows[r, pl.ds(k * L, L)] = rows[r, pl.ds(k * L, L)] * wsp

        pltpu.sync_copy(rows, acc.at[sidx2.at[j]], add=True)

    @pl.loop(0, NCHUNK)
    def _chunk(c):
        pltpu.sync_copy(dst_hbm.at[wid, c], didx2)
        pltpu.sync_copy(src_hbm.at[wid, c], sidx2)
        pltpu.sync_copy(w_hbm.at[wid, c], wbuf2)

        issue_gather(0, 0)

        @pl.loop(0, NPAIR)
        def _pair(p):
            b0 = 2 * p
            wait_gather(0)
            issue_gather(b0 + 1, 1)
            scale_scatter(b0, 0)
            wait_gather(1)
            issue_gather(b0 + 2, 0)
            scale_scatter(b0 + 1, 1)

        wait_gather(0)
        scale_scatter(CHUNKI - 1, 0)

    plsc.subcore_barrier()

    for k in range(ZCHUNKS):
        r0 = sid * ROWS_PER_TILE + k * B
        pltpu.sync_copy(acc.at[pl.ds(r0, B)], out_hbm.at[cid, pl.ds(r0, B)])


CROWS = NP // NW


@functools.partial(
    pl.kernel,
    out_type=jax.ShapeDtypeStruct((NP, D), jnp.float32),
    mesh=_mesh,
    scratch_types=[
        pltpu.VMEM((CROWS, D), jnp.float32),
        pltpu.VMEM((CROWS, D), jnp.float32),
    ],
    compiler_params=pltpu.CompilerParams(needs_layout_passes=False),
)
def _combine(p_hbm, y_hbm, a, b):
    cid = lax.axis_index("c")
    sid = lax.axis_index("s")
    wid = sid * NC + cid
    r0 = wid * CROWS

    pltpu.sync_copy(p_hbm.at[0, pl.ds(r0, CROWS)], a)
    pltpu.sync_copy(p_hbm.at[1, pl.ds(r0, CROWS)], b)

    @pl.loop(0, CROWS)
    def _row(i):
        for j in range(DJ):
            sl = pl.ds(j * L, L)
            a[i, sl] = a[i, sl] + b[i, sl]

    pltpu.sync_copy(a, y_hbm.at[pl.ds(r0, CROWS)])


def _layer(x, dst3, src3, w3):
    p = _accumulate(x, dst3, src3, w3)
    return _combine(p)


@jax.jit
def kernel(ini_embeds, edge_index, adj_values):
    src = edge_index[0].astype(jnp.int32).reshape(NW, NCHUNK, CHUNKI, B)
    dst = edge_index[1].astype(jnp.int32).reshape(NW, NCHUNK, CHUNKI, B)
    w = adj_values.reshape(NW, NCHUNK, CHUNKI, B)

    x0 = jnp.zeros((NP, D), jnp.float32).at[:N_NODES].set(ini_embeds)
    h1 = _layer(x0, dst, src, w)
    h2 = _layer(h1, dst, src, w)

    tem = jnp.concatenate([h1[:N_NODES], h2[:N_NODES]], axis=-1)
    return tem[:N_USER], tem[N_USER:]
